# baseline (speedup 1.0000x reference)
import functools

import jax
import jax.numpy as jnp
from jax import lax
from jax.experimental import pallas as pl
from jax.experimental.pallas import tpu as pltpu
from jax.experimental.pallas import tpu_sc as plsc

L = 16
NC = 2
NS = 16
NW = NC * NS

MAGIC = 12582912.0


def _build(T, F, E):
    INTER = F // 2
    ROWS = T // NW
    G = 8
    NB = ROWS // G
    WPR = INTER // 4
    NJ = INTER // L
    mesh = plsc.VectorSubcoreMesh(core_axis_name="c", subcore_axis_name="s",
                                  num_cores=NC, num_subcores=NS)

    @functools.partial(
        pl.kernel,
        out_type=[
            jax.ShapeDtypeStruct((T, WPR), jnp.int32),
            jax.ShapeDtypeStruct((T,), jnp.float32),
        ],
        mesh=mesh,
        compiler_params=pltpu.CompilerParams(needs_layout_passes=False),
        scratch_types=[
            pltpu.VMEM((E * INTER,), jnp.float32),
            pltpu.VMEM((G, F), jnp.float32),
            pltpu.VMEM((G, F), jnp.float32),
            pltpu.VMEM((ROWS,), jnp.int32),
            pltpu.VMEM((L,), jnp.int32),
            pltpu.VMEM((L,), jnp.int32),
            pltpu.VMEM((2 * INTER,), jnp.float32),
            pltpu.VMEM((G, WPR), jnp.int32),
            pltpu.VMEM((G, WPR), jnp.int32),
            pltpu.VMEM((ROWS,), jnp.float32),
            pltpu.VMEM((L,), jnp.float32),
            pltpu.SemaphoreType.DMA,
            pltpu.SemaphoreType.DMA,
            pltpu.SemaphoreType.DMA,
            pltpu.SemaphoreType.DMA,
        ],
    )
    def body(in_hbm, scale_hbm, ids_hbm, topk_hbm, beta_hbm,
             q_hbm, qs_hbm,
             scale_v, rows0, rows1, ids_v, eids0, eids1, y_v, out0, out1,
             qs_v, beta_v, sem0, sem1, semo0, semo1):
        cid = lax.axis_index("c")
        sid = lax.axis_index("s")
        wid = sid * NC + cid
        base = wid * ROWS

        pltpu.sync_copy(beta_hbm, beta_v)
        pltpu.sync_copy(scale_hbm, scale_v)
        pltpu.sync_copy(ids_hbm.at[pl.ds(base, ROWS)], ids_v)

        iota = lax.broadcasted_iota(jnp.int32, (L,), 0)
        iota4 = iota * 4
        lane0 = iota == 0
        nbeta = -beta_v[...]

        def start(n, rows_b, eids_b, sem):
            idx = ids_v.at[pl.ds(n * G, G)]
            pltpu.async_copy(in_hbm.at[idx], rows_b, sem)
            pltpu.async_copy(topk_hbm.at[idx], eids_b.at[pl.ds(0, G)], sem)

        def wait(n, rows_b, eids_b, sem):
            idx = ids_v.at[pl.ds(n * G, G)]
            pltpu.make_async_copy(in_hbm.at[idx], rows_b, sem).wait()
            pltpu.make_async_copy(topk_hbm.at[idx], eids_b.at[pl.ds(0, G)],
                                  sem).wait()

        def compute(n, rows_b, eids_b, out_b, semo):
            rbase = n * G
            @pl.when(n >= 2)
            def _():
                pltpu.make_async_copy(
                    out_b, q_hbm.at[pl.ds(base + (n - 2) * G, G)],
                    semo).wait()

            @pl.loop(0, G, step=2)
            def _row(r):
                eid0 = plsc.load_gather(eids_b, [iota * 0 + r])
                eid1 = plsc.load_gather(eids_b, [iota * 0 + (r + 1)])
                sbase0 = eid0 * INTER + iota
                sbase1 = eid1 * INTER + iota

                zero = jnp.zeros((L,), jnp.float32)

                @plsc.parallel_loop(0, NJ, unroll=4, carry=(zero, zero))
                def accs(j, a):
                    a0, a1 = a
                    col = j * L
                    g0 = rows_b[r, pl.ds(col, L)]
                    u0 = rows_b[r, pl.ds(INTER + col, L)]
                    s0 = plsc.load_gather(scale_v, [sbase0 + col])
                    e0 = jnp.exp(g0 * nbeta)
                    y0 = (g0 * u0 * s0) / (e0 + 1.0)
                    y_v[pl.ds(col, L)] = y0
                    g1 = rows_b[r + 1, pl.ds(col, L)]
                    u1 = rows_b[r + 1, pl.ds(INTER + col, L)]
                    s1 = plsc.load_gather(scale_v, [sbase1 + col])
                    e1 = jnp.exp(g1 * nbeta)
                    y1 = (g1 * u1 * s1) / (e1 + 1.0)
                    y_v[pl.ds(INTER + col, L)] = y1
                    return (jnp.maximum(a0, jnp.abs(y0)),
                            jnp.maximum(a1, jnp.abs(y1)))

                acc0, acc1 = accs
                amax0 = jnp.broadcast_to(jnp.max(acc0), (L,))
                amax1 = jnp.broadcast_to(jnp.max(acc1), (L,))
                qs0 = jnp.maximum(amax0 / 127.0, 1e-8)
                qs1 = jnp.maximum(amax1 / 127.0, 1e-8)
                inv0 = 1.0 / qs0
                inv1 = 1.0 / qs1
                plsc.store_scatter(qs_v, [iota + (rbase + r)],
                                   jnp.where(lane0, qs0, qs1),
                                   mask=iota < 2)

                @plsc.parallel_loop(0, WPR // L, unroll=4)
                def _quant(m):
                    b4 = m * (4 * L)
                    word0 = None
                    word1 = None
                    for k in range(4):
                        yv0 = plsc.load_gather(y_v, [b4 + k + iota4])
                        yv1 = plsc.load_gather(y_v, [INTER + b4 + k + iota4])
                        x0 = (yv0 * inv0 + MAGIC) - MAGIC
                        x1 = (yv1 * inv1 + MAGIC) - MAGIC
                        q0 = x0.astype(jnp.int32)
                        q1 = x1.astype(jnp.int32)
                        if k == 0:
                            w0 = q0 & 0xFF
                            w1 = q1 & 0xFF
                        elif k < 3:
                            w0 = (q0 & 0xFF) << (8 * k)
                            w1 = (q1 & 0xFF) << (8 * k)
                        else:
                            w0 = q0 << 24
                            w1 = q1 << 24
                        word0 = w0 if word0 is None else word0 | w0
                        word1 = w1 if word1 is None else word1 | w1
                    out_b[r, pl.ds(m * L, L)] = word0
                    out_b[r + 1, pl.ds(m * L, L)] = word1

            pltpu.async_copy(out_b, q_hbm.at[pl.ds(base + rbase, G)], semo)

        start(0, rows0, eids0, sem0)

        @pl.loop(0, NB, step=2)
        def _batch(b):
            start(b + 1, rows1, eids1, sem1)
            wait(b, rows0, eids0, sem0)
            compute(b, rows0, eids0, out0, semo0)

            @pl.when(b + 2 < NB)
            def _():
                start(b + 2, rows0, eids0, sem0)

            wait(b + 1, rows1, eids1, sem1)
            compute(b + 1, rows1, eids1, out1, semo1)

        pltpu.make_async_copy(
            out0, q_hbm.at[pl.ds(base + (NB - 2) * G, G)], semo0).wait()
        pltpu.make_async_copy(
            out1, q_hbm.at[pl.ds(base + (NB - 1) * G, G)], semo1).wait()
        pltpu.sync_copy(qs_v, qs_hbm.at[pl.ds(base, ROWS)])

    return body


def kernel(input, smooth_scale, sorted_token_ids, topk_indices,
           fc1_intermediate_size, beta, quant_mode):
    T, F = input.shape
    E, INTER = smooth_scale.shape
    ids = sorted_token_ids.astype(jnp.int32)
    topk = topk_indices.reshape(-1).astype(jnp.int32)
    beta_vec = jnp.full((L,), beta, jnp.float32)
    q_words, qs = _build(T, F, E)(
        input, smooth_scale.reshape(-1), ids, topk, beta_vec)
    q = lax.bitcast_convert_type(q_words, jnp.int8).reshape(T, INTER)
    return q, qs


# speedup vs baseline: 1.2607x; 1.2607x over previous
import functools

import jax
import jax.numpy as jnp
from jax import lax
from jax.experimental import pallas as pl
from jax.experimental.pallas import tpu as pltpu
from jax.experimental.pallas import tpu_sc as plsc

L = 16
NC = 2
NS = 16
NW = NC * NS

MAGIC = 12582912.0


def _build(T, F, E):
    INTER = F // 2
    ROWS = T // NW
    G = 8
    NB = ROWS // G
    WPR = INTER // 4
    NJ = INTER // L
    mesh = plsc.VectorSubcoreMesh(core_axis_name="c", subcore_axis_name="s",
                                  num_cores=NC, num_subcores=NS)

    @functools.partial(
        pl.kernel,
        out_type=[
            jax.ShapeDtypeStruct((T, WPR), jnp.int32),
            jax.ShapeDtypeStruct((T,), jnp.float32),
        ],
        mesh=mesh,
        compiler_params=pltpu.CompilerParams(needs_layout_passes=False),
        scratch_types=[
            pltpu.VMEM((E * INTER,), jnp.float32),
            pltpu.VMEM((G, F), jnp.float32),
            pltpu.VMEM((G, F), jnp.float32),
            pltpu.VMEM((ROWS,), jnp.int32),
            pltpu.VMEM((L,), jnp.int32),
            pltpu.VMEM((L,), jnp.int32),
            pltpu.VMEM((INTER,), jnp.float32),
            pltpu.VMEM((G, WPR), jnp.int32),
            pltpu.VMEM((G, WPR), jnp.int32),
            pltpu.VMEM((ROWS,), jnp.float32),
            pltpu.VMEM((L,), jnp.float32),
            pltpu.SemaphoreType.DMA,
            pltpu.SemaphoreType.DMA,
            pltpu.SemaphoreType.DMA,
            pltpu.SemaphoreType.DMA,
        ],
    )
    def body(in_hbm, scale_hbm, ids_hbm, topk_hbm, beta_hbm,
             q_hbm, qs_hbm,
             scale_v, rows0, rows1, ids_v, eids0, eids1, y_v, out0, out1,
             qs_v, beta_v, sem0, sem1, semo0, semo1):
        cid = lax.axis_index("c")
        sid = lax.axis_index("s")
        wid = sid * NC + cid
        base = wid * ROWS

        pltpu.sync_copy(beta_hbm, beta_v)
        pltpu.sync_copy(scale_hbm, scale_v)
        pltpu.sync_copy(ids_hbm.at[pl.ds(base, ROWS)], ids_v)

        iota = lax.broadcasted_iota(jnp.int32, (L,), 0)
        iota4 = iota * 4
        lane0 = iota == 0
        nbeta = -beta_v[...]

        def start(n, rows_b, eids_b, sem):
            idx = ids_v.at[pl.ds(n * G, G)]
            pltpu.async_copy(in_hbm.at[idx], rows_b, sem)
            pltpu.async_copy(topk_hbm.at[idx], eids_b.at[pl.ds(0, G)], sem)

        def wait(n, rows_b, eids_b, sem):
            idx = ids_v.at[pl.ds(n * G, G)]
            pltpu.make_async_copy(in_hbm.at[idx], rows_b, sem).wait()
            pltpu.make_async_copy(topk_hbm.at[idx], eids_b.at[pl.ds(0, G)],
                                  sem).wait()

        def compute(n, rows_b, eids_b, out_b, semo):
            rbase = n * G
            @pl.when(n >= 2)
            def _():
                pltpu.make_async_copy(
                    out_b, q_hbm.at[pl.ds(base + (n - 2) * G, G)],
                    semo).wait()

            @pl.loop(0, G)
            def _row(r):
                eid = plsc.load_gather(eids_b, [iota * 0 + r])
                sbase = eid * INTER + iota

                @plsc.parallel_loop(0, NJ, unroll=8,
                                    carry=jnp.zeros((L,), jnp.float32))
                def acc(j, a):
                    col = j * L
                    g = rows_b[r, pl.ds(col, L)]
                    u = rows_b[r, pl.ds(INTER + col, L)]
                    s = plsc.load_gather(scale_v, [sbase + col])
                    e = jnp.exp(g * nbeta)
                    y = (g * u * s) / (e + 1.0)
                    y_v[pl.ds(col, L)] = y
                    return jnp.maximum(a, jnp.abs(y))

                amax = jnp.broadcast_to(jnp.max(acc), (L,))
                qs = jnp.maximum(amax / 127.0, 1e-8)
                inv = 1.0 / qs
                plsc.store_scatter(qs_v, [iota * 0 + (rbase + r)], qs,
                                   mask=lane0)

                @plsc.parallel_loop(0, WPR // L, unroll=4)
                def _quant(m):
                    b4 = m * (4 * L)
                    word = None
                    for k in range(4):
                        yv = plsc.load_gather(y_v, [b4 + k + iota4])
                        x = (yv * inv + MAGIC) - MAGIC
                        q = x.astype(jnp.int32)
                        if k == 0:
                            w = q & 0xFF
                        elif k < 3:
                            w = (q & 0xFF) << (8 * k)
                        else:
                            w = q << 24
                        word = w if word is None else word | w
                    out_b[r, pl.ds(m * L, L)] = word

            pltpu.async_copy(out_b, q_hbm.at[pl.ds(base + rbase, G)], semo)

        start(0, rows0, eids0, sem0)

        @pl.loop(0, NB, step=2)
        def _batch(b):
            start(b + 1, rows1, eids1, sem1)
            wait(b, rows0, eids0, sem0)
            compute(b, rows0, eids0, out0, semo0)

            @pl.when(b + 2 < NB)
            def _():
                start(b + 2, rows0, eids0, sem0)

            wait(b + 1, rows1, eids1, sem1)
            compute(b + 1, rows1, eids1, out1, semo1)

        pltpu.make_async_copy(
            out0, q_hbm.at[pl.ds(base + (NB - 2) * G, G)], semo0).wait()
        pltpu.make_async_copy(
            out1, q_hbm.at[pl.ds(base + (NB - 1) * G, G)], semo1).wait()
        pltpu.sync_copy(qs_v, qs_hbm.at[pl.ds(base, ROWS)])

    return body


def kernel(input, smooth_scale, sorted_token_ids, topk_indices,
           fc1_intermediate_size, beta, quant_mode):
    T, F = input.shape
    E, INTER = smooth_scale.shape
    ids = sorted_token_ids.astype(jnp.int32)
    topk = topk_indices.reshape(-1).astype(jnp.int32)
    beta_vec = jnp.full((L,), beta, jnp.float32)
    q_words, qs = _build(T, F, E)(
        input, smooth_scale.reshape(-1), ids, topk, beta_vec)
    q32 = jnp.repeat(q_words, 4, axis=1, total_repeat_length=INTER)
    shifts = (jnp.arange(INTER, dtype=jnp.int32) % 4) * 8
    q = ((q32 >> shifts[None, :]) & 0xFF).astype(jnp.int8)
    return q, qs


# speedup vs baseline: 2.1760x; 1.7261x over previous
import functools

import jax
import jax.numpy as jnp
from jax import lax
from jax.experimental import pallas as pl
from jax.experimental.pallas import tpu as pltpu
from jax.experimental.pallas import tpu_sc as plsc

L = 16
NC = 2
NS = 16
NW = NC * NS

MAGIC = 12582912.0


def _build(T, F, E):
    INTER = F // 2
    ROWS = T // NW
    G = 8
    NB = ROWS // G
    WPR = INTER // 4
    NJ = INTER // L
    mesh = plsc.VectorSubcoreMesh(core_axis_name="c", subcore_axis_name="s",
                                  num_cores=NC, num_subcores=NS)

    @functools.partial(
        pl.kernel,
        out_type=[
            jax.ShapeDtypeStruct((T, WPR), jnp.int32),
            jax.ShapeDtypeStruct((T,), jnp.float32),
        ],
        mesh=mesh,
        compiler_params=pltpu.CompilerParams(needs_layout_passes=False),
        scratch_types=[
            pltpu.VMEM((E * INTER,), jnp.float32),
            pltpu.VMEM((G, F), jnp.float32),
            pltpu.VMEM((G, F), jnp.float32),
            pltpu.VMEM((ROWS,), jnp.int32),
            pltpu.VMEM((L,), jnp.int32),
            pltpu.VMEM((L,), jnp.int32),
            pltpu.VMEM((INTER,), jnp.float32),
            pltpu.VMEM((G, WPR), jnp.int32),
            pltpu.VMEM((G, WPR), jnp.int32),
            pltpu.VMEM((ROWS,), jnp.float32),
            pltpu.VMEM((L,), jnp.float32),
            pltpu.SemaphoreType.DMA,
            pltpu.SemaphoreType.DMA,
            pltpu.SemaphoreType.DMA,
            pltpu.SemaphoreType.DMA,
        ],
    )
    def body(in_hbm, scale_hbm, ids_hbm, topk_hbm, beta_hbm,
             q_hbm, qs_hbm,
             scale_v, rows0, rows1, ids_v, eids0, eids1, y_v, out0, out1,
             qs_v, beta_v, sem0, sem1, semo0, semo1):
        cid = lax.axis_index("c")
        sid = lax.axis_index("s")
        wid = sid * NC + cid
        base = wid * ROWS

        pltpu.sync_copy(beta_hbm, beta_v)
        pltpu.sync_copy(scale_hbm, scale_v)
        pltpu.sync_copy(ids_hbm.at[pl.ds(base, ROWS)], ids_v)

        iota = lax.broadcasted_iota(jnp.int32, (L,), 0)
        iota4 = iota * 4
        lane0 = iota == 0
        nbeta = -beta_v[...]

        def start(n, rows_b, eids_b, sem):
            idx = ids_v.at[pl.ds(n * G, G)]
            pltpu.async_copy(in_hbm.at[idx], rows_b, sem)
            pltpu.async_copy(topk_hbm.at[idx], eids_b.at[pl.ds(0, G)], sem)

        def wait(n, rows_b, eids_b, sem):
            idx = ids_v.at[pl.ds(n * G, G)]
            pltpu.make_async_copy(in_hbm.at[idx], rows_b, sem).wait()
            pltpu.make_async_copy(topk_hbm.at[idx], eids_b.at[pl.ds(0, G)],
                                  sem).wait()

        def compute(n, rows_b, eids_b, out_b, semo):
            rbase = n * G
            @pl.when(n >= 2)
            def _():
                pltpu.make_async_copy(
                    out_b, q_hbm.at[pl.ds(base + (n - 2) * G, G)],
                    semo).wait()

            @pl.loop(0, G)
            def _row(r):
                eid = plsc.load_gather(eids_b, [iota * 0 + r])
                sbase = eid * INTER + iota

                @plsc.parallel_loop(0, NJ, unroll=8,
                                    carry=jnp.zeros((L,), jnp.float32))
                def acc(j, a):
                    col = j * L
                    g = rows_b[r, pl.ds(col, L)]
                    u = rows_b[r, pl.ds(INTER + col, L)]
                    s = plsc.load_gather(scale_v, [sbase + col])
                    e = jnp.exp(g * nbeta)
                    y = (g * u * s) / (e + 1.0)
                    y_v[pl.ds(col, L)] = y
                    return jnp.maximum(a, jnp.abs(y))

                amax = jnp.broadcast_to(jnp.max(acc), (L,))
                qs = jnp.maximum(amax / 127.0, 1e-8)
                inv = 1.0 / qs
                plsc.store_scatter(qs_v, [iota * 0 + (rbase + r)], qs,
                                   mask=lane0)

                @plsc.parallel_loop(0, WPR // L, unroll=4)
                def _quant(m):
                    col = m * L
                    word = None
                    for k in range(4):
                        yv = y_v[pl.ds(k * WPR + col, L)]
                        x = (yv * inv + MAGIC) - MAGIC
                        q = x.astype(jnp.int32)
                        if k == 0:
                            w = q & 0xFF
                        elif k < 3:
                            w = (q & 0xFF) << (8 * k)
                        else:
                            w = q << 24
                        word = w if word is None else word | w
                    out_b[r, pl.ds(col, L)] = word

            pltpu.async_copy(out_b, q_hbm.at[pl.ds(base + rbase, G)], semo)

        start(0, rows0, eids0, sem0)

        @pl.loop(0, NB, step=2)
        def _batch(b):
            start(b + 1, rows1, eids1, sem1)
            wait(b, rows0, eids0, sem0)
            compute(b, rows0, eids0, out0, semo0)

            @pl.when(b + 2 < NB)
            def _():
                start(b + 2, rows0, eids0, sem0)

            wait(b + 1, rows1, eids1, sem1)
            compute(b + 1, rows1, eids1, out1, semo1)

        pltpu.make_async_copy(
            out0, q_hbm.at[pl.ds(base + (NB - 2) * G, G)], semo0).wait()
        pltpu.make_async_copy(
            out1, q_hbm.at[pl.ds(base + (NB - 1) * G, G)], semo1).wait()
        pltpu.sync_copy(qs_v, qs_hbm.at[pl.ds(base, ROWS)])

    return body


def kernel(input, smooth_scale, sorted_token_ids, topk_indices,
           fc1_intermediate_size, beta, quant_mode):
    T, F = input.shape
    E, INTER = smooth_scale.shape
    ids = sorted_token_ids.astype(jnp.int32)
    topk = topk_indices.reshape(-1).astype(jnp.int32)
    beta_vec = jnp.full((L,), beta, jnp.float32)
    q_words, qs = _build(T, F, E)(
        input, smooth_scale.reshape(-1), ids, topk, beta_vec)
    q = jnp.concatenate(
        [((q_words >> (8 * k)) & 0xFF).astype(jnp.int8) for k in range(4)],
        axis=1)
    return q, qs
